# baseline (speedup 1.0000x reference)
import functools

import jax
import jax.numpy as jnp
import numpy as np
from jax import lax
from jax.experimental import pallas as pl
from jax.experimental.pallas import tpu as pltpu
from jax.experimental.pallas import tpu_sc as plsc

NUM_SUB = 8
K = 8192
SUB_DIM = 64
DIM = NUM_SUB * SUB_DIM
NUM_CODES = 100000
R = 80
FLAT = R * NUM_SUB
NUM_CHUNKS = NUM_CODES // R
LANES = 16
TILE_ROWS = R // 8
LANE_BLKS = DIM // 128


def _perm_keys():
    keys = np.zeros((LANE_BLKS, 4, LANES), np.int32)
    for j in range(LANE_BLKS):
        for k in range(4):
            used = {}
            for m in range(LANES):
                s = m % 8
                if s in (2 * j, 2 * j + 1):
                    h = s - 2 * j
                    used[m] = 4 * k + 2 * (m // 8) + h
            park = [l for l in range(LANES)
                    if l not in set(used.values())]
            it = iter(park)
            for m in range(LANES):
                keys[j, k, m] = used.get(m, -1)
                if keys[j, k, m] < 0:
                    keys[j, k, m] = next(it)
    return keys


_KEYS = _perm_keys()


def _make_prep_kernel():
    info = plsc.get_sparse_core_info()
    nc, ns = info.num_cores, info.num_subcores
    nw = nc * ns
    t_rows = (NUM_SUB * K) // nw
    c_rows = NUM_CODES // nw
    mesh = plsc.VectorSubcoreMesh(core_axis_name="c", subcore_axis_name="s")

    @functools.partial(
        pl.kernel,
        out_type=(
            jax.ShapeDtypeStruct((NUM_SUB * K, SUB_DIM), jnp.float32),
            jax.ShapeDtypeStruct((nw, c_rows, NUM_SUB), jnp.int32),
        ),
        mesh=mesh,
        scratch_types=[
            pltpu.VMEM((t_rows // 2, SUB_DIM), jnp.float32),
            pltpu.VMEM((c_rows, NUM_SUB), jnp.int32),
        ],
        compiler_params=pltpu.CompilerParams(use_tc_tiling_on_sc=False,
                                             needs_layout_passes=False),
    )
    def prep_kernel(cent_hbm, code_hbm, table_out, code_out, trow_v, crow_v):
        wid = lax.axis_index("s") * nc + lax.axis_index("c")
        s = wid // (nw // NUM_SUB)
        k0 = (wid % (nw // NUM_SUB)) * t_rows
        for half in range(2):
            r0 = k0 + half * (t_rows // 2)
            pltpu.sync_copy(cent_hbm.at[s, pl.ds(r0, t_rows // 2)], trow_v)
            pltpu.sync_copy(trow_v,
                            table_out.at[pl.ds(s * K + r0, t_rows // 2)])
        pltpu.sync_copy(code_hbm.at[pl.ds(wid * c_rows, c_rows)], crow_v)
        pltpu.sync_copy(crow_v, code_out.at[wid])

    return prep_kernel


def _make_gather_kernel():
    info = plsc.get_sparse_core_info()
    nc, ns = info.num_cores, info.num_subcores
    nw = nc * ns
    max_mine = -(-NUM_CHUNKS // nw)
    n_pairs = -(-max_mine // 2)
    mesh = plsc.VectorSubcoreMesh(core_axis_name="c", subcore_axis_name="s")

    @functools.partial(
        pl.kernel,
        out_type=jax.ShapeDtypeStruct((NUM_CHUNKS, FLAT, SUB_DIM),
                                      jnp.float32),
        mesh=mesh,
        scratch_types=[
            pltpu.VMEM((LANE_BLKS * 4 * LANES,), jnp.int32),
            pltpu.VMEM((FLAT,), jnp.int32),
            pltpu.VMEM((FLAT,), jnp.int32),
            pltpu.VMEM((FLAT,), jnp.int32),
            pltpu.VMEM((FLAT, SUB_DIM), jnp.float32),
            pltpu.VMEM((FLAT, SUB_DIM), jnp.float32),
            pltpu.SemaphoreType.DMA,
            pltpu.SemaphoreType.DMA,
            pltpu.SemaphoreType.DMA,
            pltpu.SemaphoreType.DMA,
        ],
        compiler_params=pltpu.CompilerParams(use_tc_tiling_on_sc=False,
                                             needs_layout_passes=False),
    )
    def gather_kernel(table_hbm, code_hbm, keys_hbm, out_hbm,
                      keys_v, craw, idx0, idx1, rows0, rows1,
                      g0, g1, w0, w1):
        pltpu.sync_copy(keys_hbm, keys_v)
        wid = lax.axis_index("s") * nc + lax.axis_index("c")
        idx_b, rows_b = (idx0, idx1), (rows0, rows1)
        gsem, wsem = (g0, g1), (w0, w1)
        lane = lax.broadcasted_iota(jnp.int32, (LANES,), 0)
        offs_j = [((2 * j + (lane & 1)) * K).astype(jnp.int32)
                  for j in range(LANE_BLKS)]
        masks = [(lane >> 2) == k for k in range(4)]
        n_mine = (NUM_CHUNKS - wid + nw - 1) // nw

        def load(t, b):
            pltpu.sync_copy(code_hbm.at[wid + t * nw], craw)
            for u in range(TILE_ROWS):
                src = [craw[pl.ds(64 * u + 16 * k, LANES)] for k in range(4)]
                for j in range(LANE_BLKS):
                    acc = None
                    for k in range(4):
                        kv = keys_v[pl.ds((j * 4 + k) * LANES, LANES)]
                        _, pv = plsc.sort_key_val(kv, src[k])
                        acc = pv if acc is None else jnp.where(
                            masks[k], pv, acc)
                    idx_b[b][pl.ds((u * LANE_BLKS + j) * LANES, LANES)] = (
                        acc + offs_j[j])

            @pl.when(t >= 2)
            def _():
                pltpu.make_async_copy(
                    rows_b[b], out_hbm.at[0], wsem[b]).wait()

            pltpu.async_copy(table_hbm.at[idx_b[b]], rows_b[b], gsem[b])

        def store(t, b):
            pltpu.make_async_copy(
                table_hbm.at[idx_b[b]], rows_b[b], gsem[b]).wait()
            pltpu.async_copy(rows_b[b], out_hbm.at[wid + t * nw], wsem[b])

        load(0, 0)

        def pair(g, carry):
            t0, t1 = 2 * g, 2 * g + 1

            @pl.when(t1 < n_mine)
            def _():
                load(t1, 1)

            @pl.when(t0 < n_mine)
            def _():
                store(t0, 0)

            @pl.when(t1 + 1 < n_mine)
            def _():
                load(t1 + 1, 0)

            @pl.when(t1 < n_mine)
            def _():
                store(t1, 1)

            return carry

        lax.fori_loop(0, n_pairs, pair, 0)
        for b in (0, 1):
            pltpu.make_async_copy(
                rows_b[b], out_hbm.at[0], wsem[b]).wait()

    return gather_kernel


_prep = _make_prep_kernel()
_gather = _make_gather_kernel()


@jax.jit
def kernel(code, centroid):
    table, code_lin = _prep(centroid, code.astype(jnp.int32))
    code2 = code_lin.reshape(NUM_CHUNKS, FLAT)
    blocks = _gather(table, code2, jnp.asarray(_KEYS.reshape(-1)))
    tiled = blocks.reshape(NUM_CODES // 8, LANE_BLKS, 8, 128)
    return tiled.transpose(0, 2, 1, 3).reshape(NUM_CODES, DIM)


# speedup vs baseline: 1.0700x; 1.0700x over previous
import functools

import jax
import jax.numpy as jnp
import numpy as np
from jax import lax
from jax.experimental import pallas as pl
from jax.experimental.pallas import tpu as pltpu
from jax.experimental.pallas import tpu_sc as plsc

NUM_SUB = 8
K = 8192
SUB_DIM = 64
DIM = NUM_SUB * SUB_DIM
NUM_CODES = 100000
R = 80
FLAT = R * NUM_SUB
NUM_CHUNKS = NUM_CODES // R
LANES = 16
TILE_ROWS = R // 8
LANE_BLKS = DIM // 128


def _perm_keys():
    keys = np.zeros((LANE_BLKS, 4, LANES), np.int32)
    for j in range(LANE_BLKS):
        for k in range(4):
            used = {}
            for m in range(LANES):
                s = m % 8
                if s in (2 * j, 2 * j + 1):
                    h = s - 2 * j
                    used[m] = 4 * k + 2 * (m // 8) + h
            park = [l for l in range(LANES)
                    if l not in set(used.values())]
            it = iter(park)
            for m in range(LANES):
                keys[j, k, m] = used.get(m, -1)
                if keys[j, k, m] < 0:
                    keys[j, k, m] = next(it)
    return keys


_KEYS = _perm_keys()


def _make_gather_kernel():
    info = plsc.get_sparse_core_info()
    nc, ns = info.num_cores, info.num_subcores
    nw = nc * ns
    max_mine = -(-NUM_CHUNKS // nw)
    n_pairs = -(-max_mine // 2)
    mesh = plsc.VectorSubcoreMesh(core_axis_name="c", subcore_axis_name="s")

    @functools.partial(
        pl.kernel,
        out_type=jax.ShapeDtypeStruct((NUM_CHUNKS, FLAT, SUB_DIM),
                                      jnp.float32),
        mesh=mesh,
        scratch_types=[
            pltpu.VMEM((LANE_BLKS * 4 * LANES,), jnp.int32),
            pltpu.VMEM((FLAT,), jnp.int32),
            pltpu.VMEM((FLAT,), jnp.int32),
            pltpu.VMEM((FLAT,), jnp.int32),
            pltpu.VMEM((FLAT, SUB_DIM), jnp.float32),
            pltpu.VMEM((FLAT, SUB_DIM), jnp.float32),
            pltpu.SemaphoreType.DMA,
            pltpu.SemaphoreType.DMA,
            pltpu.SemaphoreType.DMA,
            pltpu.SemaphoreType.DMA,
        ],
        compiler_params=pltpu.CompilerParams(use_tc_tiling_on_sc=False,
                                             needs_layout_passes=False),
    )
    def gather_kernel(table_hbm, code_hbm, keys_hbm, out_hbm,
                      keys_v, craw, idx0, idx1, rows0, rows1,
                      g0, g1, w0, w1):
        pltpu.sync_copy(keys_hbm, keys_v)
        wid = lax.axis_index("s") * nc + lax.axis_index("c")
        idx_b, rows_b = (idx0, idx1), (rows0, rows1)
        gsem, wsem = (g0, g1), (w0, w1)
        lane = lax.broadcasted_iota(jnp.int32, (LANES,), 0)
        offs_j = [((2 * j + (lane & 1)) * K).astype(jnp.int32)
                  for j in range(LANE_BLKS)]
        masks = [(lane >> 2) == k for k in range(4)]
        n_mine = (NUM_CHUNKS - wid + nw - 1) // nw

        def load(t, b):
            pltpu.sync_copy(
                code_hbm.at[pl.ds((wid + t * nw) * FLAT, FLAT)], craw)
            for u in range(TILE_ROWS):
                src = [craw[pl.ds(64 * u + 16 * k, LANES)] for k in range(4)]
                for j in range(LANE_BLKS):
                    acc = None
                    for k in range(4):
                        kv = keys_v[pl.ds((j * 4 + k) * LANES, LANES)]
                        _, pv = plsc.sort_key_val(kv, src[k])
                        acc = pv if acc is None else jnp.where(
                            masks[k], pv, acc)
                    idx_b[b][pl.ds((u * LANE_BLKS + j) * LANES, LANES)] = (
                        acc + offs_j[j])

            @pl.when(t >= 2)
            def _():
                pltpu.make_async_copy(
                    rows_b[b], out_hbm.at[0], wsem[b]).wait()

            pltpu.async_copy(table_hbm.at[idx_b[b]], rows_b[b], gsem[b])

        def store(t, b):
            pltpu.make_async_copy(
                table_hbm.at[idx_b[b]], rows_b[b], gsem[b]).wait()
            pltpu.async_copy(rows_b[b], out_hbm.at[wid + t * nw], wsem[b])

        load(0, 0)

        def pair(g, carry):
            t0, t1 = 2 * g, 2 * g + 1

            @pl.when(t1 < n_mine)
            def _():
                load(t1, 1)

            @pl.when(t0 < n_mine)
            def _():
                store(t0, 0)

            @pl.when(t1 + 1 < n_mine)
            def _():
                load(t1 + 1, 0)

            @pl.when(t1 < n_mine)
            def _():
                store(t1, 1)

            return carry

        lax.fori_loop(0, n_pairs, pair, 0)
        for b in (0, 1):
            pltpu.make_async_copy(
                rows_b[b], out_hbm.at[0], wsem[b]).wait()

    return gather_kernel


_gather = _make_gather_kernel()


@jax.jit
def kernel(code, centroid):
    table = centroid.reshape(NUM_SUB * K, SUB_DIM)
    code1d = code.astype(jnp.int32).reshape(NUM_CHUNKS * FLAT)
    blocks = _gather(table, code1d, jnp.asarray(_KEYS.reshape(-1)))
    tiled = blocks.reshape(NUM_CODES // 8, LANE_BLKS, 8, 128)
    return tiled.transpose(0, 2, 1, 3).reshape(NUM_CODES, DIM)


# speedup vs baseline: 1.3604x; 1.2713x over previous
import functools

import jax
import jax.numpy as jnp
import numpy as np
from jax import lax
from jax.experimental import pallas as pl
from jax.experimental.pallas import tpu as pltpu
from jax.experimental.pallas import tpu_sc as plsc

NUM_SUB = 8
K = 8192
SUB_DIM = 64
DIM = NUM_SUB * SUB_DIM
NUM_CODES = 100000
R = 80
FLAT = R * NUM_SUB
NUM_CHUNKS = NUM_CODES // R
LANES = 16
TILE_ROWS = R // 8
LANE_BLKS = DIM // 128


def _perm_keys():
    keys = np.zeros((LANE_BLKS, 4, LANES), np.int32)
    for j in range(LANE_BLKS):
        for k in range(4):
            used = {}
            for m in range(LANES):
                s = m % 8
                if s in (2 * j, 2 * j + 1):
                    h = s - 2 * j
                    used[m] = 4 * k + 2 * (m // 8) + h
            park = [l for l in range(LANES)
                    if l not in set(used.values())]
            it = iter(park)
            for m in range(LANES):
                keys[j, k, m] = used.get(m, -1)
                if keys[j, k, m] < 0:
                    keys[j, k, m] = next(it)
    return keys


_KEYS = _perm_keys()


def _make_gather_kernel():
    info = plsc.get_sparse_core_info()
    nc, ns = info.num_cores, info.num_subcores
    nw = nc * ns
    max_mine = -(-NUM_CHUNKS // nw)
    n_pairs = -(-max_mine // 2)
    mesh = plsc.VectorSubcoreMesh(core_axis_name="c", subcore_axis_name="s")

    @functools.partial(
        pl.kernel,
        out_type=jax.ShapeDtypeStruct((NUM_CHUNKS, FLAT, SUB_DIM),
                                      jnp.float32),
        mesh=mesh,
        scratch_types=[
            pltpu.VMEM((NUM_SUB, 96), jnp.int32),
            pltpu.VMEM((FLAT,), jnp.int32),
            pltpu.VMEM((FLAT,), jnp.int32),
            pltpu.VMEM((FLAT, SUB_DIM), jnp.float32),
            pltpu.VMEM((FLAT, SUB_DIM), jnp.float32),
            pltpu.SemaphoreType.DMA,
            pltpu.SemaphoreType.DMA,
            pltpu.SemaphoreType.DMA,
            pltpu.SemaphoreType.DMA,
        ],
        compiler_params=pltpu.CompilerParams(use_tc_tiling_on_sc=False,
                                             needs_layout_passes=False),
    )
    def gather_kernel(table_hbm, code_hbm, out_hbm,
                      craw, idx0, idx1, rows0, rows1,
                      g0, g1, w0, w1):
        wid = lax.axis_index("s") * nc + lax.axis_index("c")
        idx_b, rows_b = (idx0, idx1), (rows0, rows1)
        gsem, wsem = (g0, g1), (w0, w1)
        lane = lax.broadcasted_iota(jnp.int32, (LANES,), 0)
        offs_j = [((2 * j + (lane & 1)) * K).astype(jnp.int32)
                  for j in range(LANE_BLKS)]
        ka = ((lane * 2) & 15) + (lane >> 3)
        kb = ((lane * 2) & 15) + 1 - (lane >> 3)
        odd = (lane & 1) == 1
        n_mine = (NUM_CHUNKS - wid + nw - 1) // nw

        def load(t, b):
            n0 = (wid + t * nw) * R
            pltpu.sync_copy(code_hbm.at[:, pl.ds(n0, R)],
                            craw.at[:, pl.ds(0, R)])
            for u in range(TILE_ROWS):
                for j in range(LANE_BLKS):
                    va = craw[2 * j, pl.ds(8 * u, LANES)]
                    vb = craw[2 * j + 1, pl.ds(8 * u, LANES)]
                    _, sa = plsc.sort_key_val(ka, va)
                    _, sb = plsc.sort_key_val(kb, vb)
                    idx_b[b][pl.ds((u * LANE_BLKS + j) * LANES, LANES)] = (
                        jnp.where(odd, sb, sa) + offs_j[j])

            @pl.when(t >= 2)
            def _():
                pltpu.make_async_copy(
                    rows_b[b], out_hbm.at[0], wsem[b]).wait()

            pltpu.async_copy(table_hbm.at[idx_b[b]], rows_b[b], gsem[b])

        def store(t, b):
            pltpu.make_async_copy(
                table_hbm.at[idx_b[b]], rows_b[b], gsem[b]).wait()
            pltpu.async_copy(rows_b[b], out_hbm.at[wid + t * nw], wsem[b])

        load(0, 0)

        def pair(g, carry):
            t0, t1 = 2 * g, 2 * g + 1

            @pl.when(t1 < n_mine)
            def _():
                load(t1, 1)

            @pl.when(t0 < n_mine)
            def _():
                store(t0, 0)

            @pl.when(t1 + 1 < n_mine)
            def _():
                load(t1 + 1, 0)

            @pl.when(t1 < n_mine)
            def _():
                store(t1, 1)

            return carry

        lax.fori_loop(0, n_pairs, pair, 0)
        for b in (0, 1):
            pltpu.make_async_copy(
                rows_b[b], out_hbm.at[0], wsem[b]).wait()

    return gather_kernel


_gather = _make_gather_kernel()


@jax.jit
def kernel(code, centroid):
    table = centroid.reshape(NUM_SUB * K, SUB_DIM)
    code_t = code.astype(jnp.int32).T
    blocks = _gather(table, code_t)
    tiled = blocks.reshape(NUM_CODES // 8, LANE_BLKS, 8, 128)
    return tiled.transpose(0, 2, 1, 3).reshape(NUM_CODES, DIM)


# speedup vs baseline: 1.3609x; 1.0004x over previous
import functools

import jax
import jax.numpy as jnp
from jax import lax
from jax.experimental import pallas as pl
from jax.experimental.pallas import tpu as pltpu
from jax.experimental.pallas import tpu_sc as plsc

NUM_SUB = 8
K = 8192
SUB_DIM = 64
DIM = NUM_SUB * SUB_DIM
NUM_CODES = 100000
R = 80
FLAT = R * NUM_SUB
NUM_CHUNKS = NUM_CODES // R
LANES = 16
TILE_ROWS = R // 8
LANE_BLKS = DIM // 128


def _make_gather_kernel():
    info = plsc.get_sparse_core_info()
    nc, ns = info.num_cores, info.num_subcores
    nw = nc * ns
    max_mine = -(-NUM_CHUNKS // nw)
    n_pairs = -(-max_mine // 2)
    mesh = plsc.VectorSubcoreMesh(core_axis_name="c", subcore_axis_name="s")

    @functools.partial(
        pl.kernel,
        out_type=jax.ShapeDtypeStruct((NUM_CHUNKS, FLAT, SUB_DIM),
                                      jnp.float32),
        mesh=mesh,
        scratch_types=[
            pltpu.VMEM((NUM_SUB, 96), jnp.int32),
            pltpu.VMEM((FLAT,), jnp.int32),
            pltpu.VMEM((FLAT,), jnp.int32),
            pltpu.VMEM((FLAT, SUB_DIM), jnp.float32),
            pltpu.VMEM((FLAT, SUB_DIM), jnp.float32),
            pltpu.SemaphoreType.DMA,
            pltpu.SemaphoreType.DMA,
            pltpu.SemaphoreType.DMA,
            pltpu.SemaphoreType.DMA,
        ],
        compiler_params=pltpu.CompilerParams(use_tc_tiling_on_sc=False,
                                             needs_layout_passes=False),
    )
    def gather_kernel(table_hbm, code_hbm, out_hbm,
                      craw, idx0, idx1, rows0, rows1,
                      g0, g1, w0, w1):
        wid = lax.axis_index("s") * nc + lax.axis_index("c")
        idx_b, rows_b = (idx0, idx1), (rows0, rows1)
        gsem, wsem = (g0, g1), (w0, w1)
        lane = lax.broadcasted_iota(jnp.int32, (LANES,), 0)
        offs_j = [((2 * j + (lane & 1)) * K).astype(jnp.int32)
                  for j in range(LANE_BLKS)]
        ka = ((lane * 2) & 15) + (lane >> 3)
        kb = ((lane * 2) & 15) + 1 - (lane >> 3)
        odd = (lane & 1) == 1
        n_mine = (NUM_CHUNKS - wid + nw - 1) // nw

        def load(t, b):
            n0 = (wid + t * nw) * R
            pltpu.sync_copy(code_hbm.at[:, pl.ds(n0, R)],
                            craw.at[:, pl.ds(0, R)])
            for u in range(TILE_ROWS):
                for j in range(LANE_BLKS):
                    va = craw[2 * j, pl.ds(8 * u, LANES)]
                    vb = craw[2 * j + 1, pl.ds(8 * u, LANES)]
                    _, sa = plsc.sort_key_val(ka, va)
                    _, sb = plsc.sort_key_val(kb, vb)
                    idx_b[b][pl.ds((u * LANE_BLKS + j) * LANES, LANES)] = (
                        jnp.where(odd, sb, sa) + offs_j[j])

            @pl.when(t >= 2)
            def _():
                pltpu.make_async_copy(
                    rows_b[b], out_hbm.at[0], wsem[b]).wait()

            pltpu.async_copy(table_hbm.at[idx_b[b]], rows_b[b], gsem[b])

        def store(t, b):
            pltpu.make_async_copy(
                table_hbm.at[idx_b[b]], rows_b[b], gsem[b]).wait()
            pltpu.async_copy(rows_b[b], out_hbm.at[wid + t * nw], wsem[b])

        load(0, 0)

        def pair(g, carry):
            t0, t1 = 2 * g, 2 * g + 1

            @pl.when(t1 < n_mine)
            def _():
                load(t1, 1)

            @pl.when(t0 < n_mine)
            def _():
                store(t0, 0)

            @pl.when(t1 + 1 < n_mine)
            def _():
                load(t1 + 1, 0)

            @pl.when(t1 < n_mine)
            def _():
                store(t1, 1)

            return carry

        lax.fori_loop(0, n_pairs, pair, 0)
        for b in (0, 1):
            pltpu.make_async_copy(
                rows_b[b], out_hbm.at[0], wsem[b]).wait()

    return gather_kernel


_gather = _make_gather_kernel()


@jax.jit
def kernel(code, centroid):
    table = centroid.reshape(NUM_SUB * K, SUB_DIM)
    code_t = code.astype(jnp.int32).T
    blocks = _gather(table, code_t)
    tiled = blocks.reshape(NUM_CODES // 8, LANE_BLKS, 8, 128)
    return tiled.transpose(0, 2, 1, 3).reshape(NUM_CODES, DIM)
